# baseline (speedup 1.0000x reference)
import functools

import jax
import jax.numpy as jnp
from jax import lax
from jax.experimental import pallas as pl
from jax.experimental.pallas import tpu as pltpu
from jax.experimental.pallas import tpu_sc as plsc

_MARGIN = 1.0
_EPS = 1e-6
_NUM_NODES = 10000
_NUM_EDGES = 160000
_SEQ_LEN = 4
_DIM = 256

_NC = 2
_NS = 16
_NW = _NC * _NS
_L = 16

_DIMW = _DIM // 2
_CHUNK = 64
_NCHUNKS = _NUM_EDGES // _CHUNK
_GROUPS = _CHUNK // _L
_KPW = 80


def _rsqrt_nr(x):
    i = plsc.bitcast(x, jnp.int32)
    i = 0x5F3759DF - (i >> 1)
    y = plsc.bitcast(i, jnp.float32)
    for _ in range(3):
        y = y * (1.5 - 0.5 * x * y * y)
    return y


def _sc_kernel(table_hbm, edge_hbm, out_hbm,
               aidx2, pidx2, ga2, gp2, gn2,
               arows2, prows2, nrows2, loss_v,
               sem_idx, sem_g):
    wid = lax.axis_index("s") * _NC + lax.axis_index("c")
    lanes = lax.iota(jnp.int32, _L)

    def base_of(kc):
        cid = jnp.minimum(wid + kc * _NW, _NCHUNKS - 1)
        return cid * _CHUNK

    def fire_idx(kc, b):
        base = base_of(kc)
        pltpu.async_copy(edge_hbm.at[0, pl.ds(base, _CHUNK)],
                         aidx2.at[b], sem_idx.at[b])
        pltpu.async_copy(edge_hbm.at[1, pl.ds(base, _CHUNK)],
                         pidx2.at[b], sem_idx.at[b])

    def wait_idx(b):
        pltpu.make_async_copy(edge_hbm.at[0, pl.ds(0, _CHUNK)],
                              aidx2.at[b], sem_idx.at[b]).wait()
        pltpu.make_async_copy(edge_hbm.at[1, pl.ds(0, _CHUNK)],
                              pidx2.at[b], sem_idx.at[b]).wait()

    def build_fire(b):
        wait_idx(b)
        aidx_v, pidx_v = aidx2.at[b], pidx2.at[b]
        ga_v, gp_v, gn_v = ga2.at[b], gp2.at[b], gn2.at[b]
        for g in range(_GROUPS):
            sl = pl.ds(g * _L, _L)
            a16 = aidx_v[sl]
            p16 = pidx_v[sl]
            h = (a16.astype(jnp.uint32) * jnp.uint32(2654435761)
                 + jnp.uint32(12345))
            n16 = (h % jnp.uint32(_NUM_NODES)).astype(jnp.int32)
            for _ in range(3):
                bad = (n16 == a16) | (n16 == p16)
                n16 = jnp.where(bad, (n16 + 1) % _NUM_NODES, n16)
            ga_v[sl] = a16
            gp_v[sl] = p16
            gn_v[sl] = n16
        pltpu.async_copy(table_hbm.at[ga2.at[b]], arows2.at[b], sem_g.at[b])
        pltpu.async_copy(table_hbm.at[gp2.at[b]], prows2.at[b], sem_g.at[b])
        pltpu.async_copy(table_hbm.at[gn2.at[b]], nrows2.at[b], sem_g.at[b])

    def wait_gathers(b):
        pltpu.make_async_copy(table_hbm.at[ga2.at[b]],
                              arows2.at[b], sem_g.at[b]).wait()
        pltpu.make_async_copy(table_hbm.at[gp2.at[b]],
                              prows2.at[b], sem_g.at[b]).wait()
        pltpu.make_async_copy(table_hbm.at[gn2.at[b]],
                              nrows2.at[b], sem_g.at[b]).wait()

    def compute(kc, b, loss):
        valid = ((wid + kc * _NW) < _NCHUNKS).astype(jnp.float32)
        vmask = lax.broadcast(valid, (_L,))
        arows_v, prows_v, nrows_v = arows2.at[b], prows2.at[b], nrows2.at[b]
        for g in range(_GROUPS):
            rows = lanes + g * _L
            zero = jnp.zeros((_L,), jnp.float32)
            cols0 = tuple((lanes + j) & 0xF for j in range(_L))

            @plsc.parallel_loop(0, _DIMW // _L, carry=(zero, zero, zero, zero) + cols0)
            def dim_body(t, acc):
                accp0, accp1, accn0, accn1 = acc[:4]
                cols = list(acc[4:])
                for j in range(_L):
                    col = cols[j]
                    vaw = plsc.load_gather(arows_v, [rows, col])
                    vpw = plsc.load_gather(prows_v, [rows, col])
                    vnw = plsc.load_gather(nrows_v, [rows, col])
                    va = plsc.bitcast(vaw, jnp.bfloat16)
                    vp = plsc.bitcast(vpw, jnp.bfloat16)
                    vn = plsc.bitcast(vnw, jnp.bfloat16)
                    va_e = va + jnp.bfloat16(_EPS)
                    dpb = plsc.bitcast(va_e - vp, jnp.int32)
                    dnb = plsc.bitcast(va_e - vn, jnp.int32)
                    dplo = plsc.bitcast(dpb << 16, jnp.float32)
                    dphi = plsc.bitcast(dpb & jnp.int32(-65536), jnp.float32)
                    dnlo = plsc.bitcast(dnb << 16, jnp.float32)
                    dnhi = plsc.bitcast(dnb & jnp.int32(-65536), jnp.float32)
                    accp0 = accp0 + dplo * dplo
                    accp1 = accp1 + dphi * dphi
                    accn0 = accn0 + dnlo * dnlo
                    accn1 = accn1 + dnhi * dnhi
                    cols[j] = col + _L
                return (accp0, accp1, accn0, accn1) + tuple(cols)

            accp0, accp1, accn0, accn1 = dim_body[:4]
            accp = accp0 + accp1
            accn = accn0 + accn1
            pos_d = accp * _rsqrt_nr(accp)
            neg_d = accn * _rsqrt_nr(accn)
            loss = loss + jnp.maximum(pos_d - neg_d + _MARGIN, 0.0) * vmask
        return loss

    fire_idx(0, 0)
    build_fire(0)
    fire_idx(1, 1)

    def pair_body(k2, loss):
        build_fire(1)
        @pl.when(k2 < _KPW // 2 - 1)
        def _():
            fire_idx(2 * k2 + 2, 0)
        wait_gathers(0)
        loss = compute(2 * k2, 0, loss)

        @pl.when(k2 < _KPW // 2 - 1)
        def _():
            build_fire(0)
            fire_idx(2 * k2 + 3, 1)
        wait_gathers(1)
        loss = compute(2 * k2 + 1, 1, loss)
        return loss

    loss = lax.fori_loop(0, _KPW // 2, pair_body,
                         jnp.zeros((_L,), jnp.float32))
    loss_v[...] = loss
    pltpu.sync_copy(loss_v, out_hbm.at[wid])


def _make_sc_call():
    mesh = plsc.VectorSubcoreMesh(core_axis_name="c", subcore_axis_name="s")
    return functools.partial(
        pl.kernel,
        mesh=mesh,
        out_type=jax.ShapeDtypeStruct((_NW, _L), jnp.float32),
        compiler_params=pltpu.CompilerParams(
            use_tc_tiling_on_sc=False, needs_layout_passes=False),
        scratch_types=[
            pltpu.VMEM((2, _CHUNK), jnp.int32),
            pltpu.VMEM((2, _CHUNK), jnp.int32),
            pltpu.VMEM((2, _CHUNK), jnp.int32),
            pltpu.VMEM((2, _CHUNK), jnp.int32),
            pltpu.VMEM((2, _CHUNK), jnp.int32),
            pltpu.VMEM((2, _CHUNK, _DIMW), jnp.int32),
            pltpu.VMEM((2, _CHUNK, _DIMW), jnp.int32),
            pltpu.VMEM((2, _CHUNK, _DIMW), jnp.int32),
            pltpu.VMEM((_L,), jnp.float32),
            pltpu.SemaphoreType.DMA((2,)),
            pltpu.SemaphoreType.DMA((2,)),
        ],
    )(_sc_kernel)


def _mean_kernel(x_ref, o_ref):
    o_ref[...] = jnp.sum(x_ref[...], axis=(0, 1), keepdims=True) * (
        1.0 / _NUM_EDGES)


@jax.jit
def kernel(embeddings, edge_index):
    emb_last = embeddings[:, -1, :].astype(jnp.bfloat16)
    table = jax.lax.bitcast_convert_type(
        emb_last.reshape(_NUM_NODES, _DIMW, 2), jnp.int32)
    partials = _make_sc_call()(table, edge_index)
    loss = pl.pallas_call(
        _mean_kernel,
        out_shape=jax.ShapeDtypeStruct((1, 1), jnp.float32),
    )(partials)
    return loss[0, 0]


# speedup vs baseline: 1.4992x; 1.4992x over previous
import functools

import jax
import jax.numpy as jnp
from jax import lax
from jax.experimental import pallas as pl
from jax.experimental.pallas import tpu as pltpu
from jax.experimental.pallas import tpu_sc as plsc

_MARGIN = 1.0
_EPS = 1e-6
_NUM_NODES = 10000
_NUM_EDGES = 160000
_SEQ_LEN = 4
_DIM = 256

_NC = 2
_NS = 16
_NW = _NC * _NS
_L = 16

_CHUNK = 80
_NCHUNKS = _NUM_EDGES // _CHUNK
_GROUPS = _CHUNK // _L
_KPW = 64


def _rsqrt_nr(x):
    i = plsc.bitcast(x, jnp.int32)
    i = 0x5F3759DF - (i >> 1)
    y = plsc.bitcast(i, jnp.float32)
    for _ in range(3):
        y = y * (1.5 - 0.5 * x * y * y)
    return y


def _sc_kernel(table_hbm, edge_hbm, out_hbm,
               aidx2, pidx2, ga2, gp2, gn2,
               arows2, prows2, nrows2, loss_v,
               sem_idx, sem_g):
    wid = lax.axis_index("s") * _NC + lax.axis_index("c")
    lanes = lax.iota(jnp.int32, _L)

    def base_of(kc):
        cid = jnp.minimum(wid + kc * _NW, _NCHUNKS - 1)
        return cid * _CHUNK

    def fire_idx(kc, b):
        base = base_of(kc)
        pltpu.async_copy(edge_hbm.at[0, pl.ds(base, _CHUNK)],
                         aidx2.at[b], sem_idx.at[b])
        pltpu.async_copy(edge_hbm.at[1, pl.ds(base, _CHUNK)],
                         pidx2.at[b], sem_idx.at[b])

    def wait_idx(b):
        pltpu.make_async_copy(edge_hbm.at[0, pl.ds(0, _CHUNK)],
                              aidx2.at[b], sem_idx.at[b]).wait()
        pltpu.make_async_copy(edge_hbm.at[1, pl.ds(0, _CHUNK)],
                              pidx2.at[b], sem_idx.at[b]).wait()

    def build_fire(b):
        wait_idx(b)
        aidx_v, pidx_v = aidx2.at[b], pidx2.at[b]
        ga_v, gp_v, gn_v = ga2.at[b], gp2.at[b], gn2.at[b]
        for g in range(_GROUPS):
            sl = pl.ds(g * _L, _L)
            a16 = aidx_v[sl]
            p16 = pidx_v[sl]
            h = (a16.astype(jnp.uint32) * jnp.uint32(2654435761)
                 + jnp.uint32(12345))
            n16 = (h % jnp.uint32(_NUM_NODES)).astype(jnp.int32)
            for _ in range(3):
                bad = (n16 == a16) | (n16 == p16)
                n16 = jnp.where(bad, (n16 + 1) % _NUM_NODES, n16)
            ga_v[sl] = a16 * _SEQ_LEN + (_SEQ_LEN - 1)
            gp_v[sl] = p16 * _SEQ_LEN + (_SEQ_LEN - 1)
            gn_v[sl] = n16 * _SEQ_LEN + (_SEQ_LEN - 1)
        pltpu.async_copy(table_hbm.at[ga2.at[b]], arows2.at[b], sem_g.at[b])
        pltpu.async_copy(table_hbm.at[gp2.at[b]], prows2.at[b], sem_g.at[b])
        pltpu.async_copy(table_hbm.at[gn2.at[b]], nrows2.at[b], sem_g.at[b])

    def wait_gathers(b):
        pltpu.make_async_copy(table_hbm.at[ga2.at[b]],
                              arows2.at[b], sem_g.at[b]).wait()
        pltpu.make_async_copy(table_hbm.at[gp2.at[b]],
                              prows2.at[b], sem_g.at[b]).wait()
        pltpu.make_async_copy(table_hbm.at[gn2.at[b]],
                              nrows2.at[b], sem_g.at[b]).wait()

    def compute(kc, b, loss):
        valid = ((wid + kc * _NW) < _NCHUNKS).astype(jnp.float32)
        vmask = lax.broadcast(valid, (_L,))
        arows_v, prows_v, nrows_v = arows2.at[b], prows2.at[b], nrows2.at[b]
        for g in range(_GROUPS):
            rows = lanes + g * _L
            zero = jnp.zeros((_L,), jnp.float32)
            cols0 = tuple((lanes + j) & 0xF for j in range(_L))

            @plsc.parallel_loop(0, _DIM // _L, carry=(zero, zero, zero, zero) + cols0)
            def dim_body(t, acc):
                accp0, accp1, accn0, accn1 = acc[:4]
                cols = list(acc[4:])
                for j in range(_L):
                    col = cols[j]
                    va = plsc.load_gather(arows_v, [rows, col])
                    vp = plsc.load_gather(prows_v, [rows, col])
                    vn = plsc.load_gather(nrows_v, [rows, col])
                    va_e = va + _EPS
                    dp = va_e - vp
                    dn = va_e - vn
                    if j % 2 == 0:
                        accp0 = accp0 + dp * dp
                        accn0 = accn0 + dn * dn
                    else:
                        accp1 = accp1 + dp * dp
                        accn1 = accn1 + dn * dn
                    cols[j] = col + _L
                return (accp0, accp1, accn0, accn1) + tuple(cols)

            accp0, accp1, accn0, accn1 = dim_body[:4]
            accp = accp0 + accp1
            accn = accn0 + accn1
            pos_d = accp * _rsqrt_nr(accp)
            neg_d = accn * _rsqrt_nr(accn)
            loss = loss + jnp.maximum(pos_d - neg_d + _MARGIN, 0.0) * vmask
        return loss

    fire_idx(0, 0)
    build_fire(0)
    fire_idx(1, 1)

    def pair_body(k2, loss):
        build_fire(1)
        @pl.when(k2 < _KPW // 2 - 1)
        def _():
            fire_idx(2 * k2 + 2, 0)
        wait_gathers(0)
        loss = compute(2 * k2, 0, loss)

        @pl.when(k2 < _KPW // 2 - 1)
        def _():
            build_fire(0)
            fire_idx(2 * k2 + 3, 1)
        wait_gathers(1)
        loss = compute(2 * k2 + 1, 1, loss)
        return loss

    loss = lax.fori_loop(0, _KPW // 2, pair_body,
                         jnp.zeros((_L,), jnp.float32))
    loss_v[...] = loss
    pltpu.sync_copy(loss_v, out_hbm.at[wid])


def _make_sc_call():
    mesh = plsc.VectorSubcoreMesh(core_axis_name="c", subcore_axis_name="s")
    return functools.partial(
        pl.kernel,
        mesh=mesh,
        out_type=jax.ShapeDtypeStruct((_NW, _L), jnp.float32),
        compiler_params=pltpu.CompilerParams(
            use_tc_tiling_on_sc=False, needs_layout_passes=False),
        scratch_types=[
            pltpu.VMEM((2, _CHUNK), jnp.int32),
            pltpu.VMEM((2, _CHUNK), jnp.int32),
            pltpu.VMEM((2, _CHUNK), jnp.int32),
            pltpu.VMEM((2, _CHUNK), jnp.int32),
            pltpu.VMEM((2, _CHUNK), jnp.int32),
            pltpu.VMEM((2, _CHUNK, _DIM), jnp.float32),
            pltpu.VMEM((2, _CHUNK, _DIM), jnp.float32),
            pltpu.VMEM((2, _CHUNK, _DIM), jnp.float32),
            pltpu.VMEM((_L,), jnp.float32),
            pltpu.SemaphoreType.DMA((2,)),
            pltpu.SemaphoreType.DMA((2,)),
        ],
    )(_sc_kernel)


def _mean_kernel(x_ref, o_ref):
    o_ref[...] = jnp.sum(x_ref[...], axis=(0, 1), keepdims=True) * (
        1.0 / _NUM_EDGES)


@jax.jit
def kernel(embeddings, edge_index):
    table = embeddings.reshape(_NUM_NODES * _SEQ_LEN, _DIM)
    partials = _make_sc_call()(table, edge_index)
    loss = pl.pallas_call(
        _mean_kernel,
        out_shape=jax.ShapeDtypeStruct((1, 1), jnp.float32),
    )(partials)
    return loss[0, 0]
